# baseline (speedup 1.0000x reference)
import jax
import jax.numpy as jnp
from jax import lax
from jax.experimental import pallas as pl
from jax.experimental.pallas import tpu as pltpu
from jax.experimental.pallas import tpu_sc as plsc

_N_NODES = 10000
_D = 128
_E = 320000
_NC = 2
_NS = 16
_NW = _NC * _NS
_EPW = _E // _NW
_C = 80
_NCHUNK = _EPW // _C
_G = _C // 16
_NIDX = 2


def _dot_chunk(sb, db, ob, dotsbuf, last_lane):
    for g in range(_G):
        for e in range(16):
            ee = g * 16 + e
            p = sb[ee, pl.ds(0, 16)] * db[ee, pl.ds(0, 16)]
            for c in range(1, _D // 16):
                p = p + sb[ee, pl.ds(c * 16, 16)] * db[ee, pl.ds(c * 16, 16)]
            cum = plsc.cumsum(p)
            plsc.store_scatter(dotsbuf, [jnp.full((16,), ee, jnp.int32)], cum,
                               mask=last_lane)
        v = dotsbuf[pl.ds(g * 16, 16)]
        ob[pl.ds(g * 16, 16)] = 1.0 / (1.0 + jnp.exp(-v))


def _edge_kernel(x_hbm, edge_hbm, out_hbm, x_s,
                 sb0, db0, sb1, db1, ob0, ob1, dotsbuf,
                 is0, id0, is1, id1,
                 ss0, sd0, ss1, sd1, so0, so1,
                 qs0, qd0, qs1, qd1):
    cid = lax.axis_index("c")
    sid = lax.axis_index("s")
    wid = sid * _NC + cid
    base = wid * _EPW

    rows_per = 640

    @pl.when(sid < _NS - 1)
    def _():
        pltpu.sync_copy(x_hbm.at[pl.ds(sid * rows_per, rows_per)],
                        x_s.at[pl.ds(sid * rows_per, rows_per)])

    @pl.when(sid == _NS - 1)
    def _():
        last = (_NS - 1) * rows_per
        pltpu.sync_copy(x_hbm.at[pl.ds(last, _N_NODES - last)],
                        x_s.at[pl.ds(last, _N_NODES - last)])

    plsc.subcore_barrier()

    last_lane = lax.broadcasted_iota(jnp.int32, (16,), 0) == 15

    idx_ring = ((is0, id0, qs0, qd0), (is1, id1, qs1, qd1))
    row_slots = ((sb0, db0, ss0, sd0), (sb1, db1, ss1, sd1))
    out_slots = ((ob0, so0), (ob1, so1))

    def start_idx(cc, j):
        ib_s, ib_d, q_s, q_d = idx_ring[j]
        o = base + cc * _C
        pltpu.async_copy(edge_hbm.at[pl.ds(o, _C)], ib_s, q_s)
        pltpu.async_copy(edge_hbm.at[pl.ds(_E + o, _C)], ib_d, q_d)

    def wait_idx(j):
        ib_s, ib_d, q_s, q_d = idx_ring[j]
        pltpu.make_async_copy(edge_hbm.at[pl.ds(0, _C)], ib_s, q_s).wait()
        pltpu.make_async_copy(edge_hbm.at[pl.ds(0, _C)], ib_d, q_d).wait()

    def start_rows(j, par):
        ib_s, ib_d, _, _ = idx_ring[j]
        sb, db, ss, sd = row_slots[par]
        pltpu.async_copy(x_s.at[ib_s], sb, ss)
        pltpu.async_copy(x_s.at[ib_d], db, sd)

    def wait_rows(par):
        sb, db, ss, sd = row_slots[par]
        pltpu.make_async_copy(x_s.at[pl.ds(0, _C)], sb, ss).wait()
        pltpu.make_async_copy(x_s.at[pl.ds(0, _C)], db, sd).wait()

    def start_out(cc, par):
        ob, so = out_slots[par]
        pltpu.async_copy(ob, out_hbm.at[pl.ds(base + cc * _C, _C)], so)

    def wait_out(par):
        ob, so = out_slots[par]
        pltpu.make_async_copy(ob, out_hbm.at[pl.ds(0, _C)], so).wait()

    for j in range(_NIDX):
        start_idx(j, j)
    wait_idx(0)
    start_rows(0, 0)
    wait_idx(1)
    start_rows(1, 1)

    @pl.loop(0, _NCHUNK - 1, step=2)
    def _pair(c):
        for par in range(2):
            cc = c + par
            wait_rows(par)

            @pl.when(cc + _NIDX < _NCHUNK)
            def _(cc=cc, par=par):
                start_idx(cc + _NIDX, par)

            @pl.when(cc >= 2)
            def _(par=par):
                wait_out(par)

            ob, _ = out_slots[par]
            _dot_chunk(*row_slots[par][:2], ob, dotsbuf, last_lane)
            start_out(cc, par)

            @pl.when(cc + 2 < _NCHUNK)
            def _(cc=cc, par=par):
                wait_idx(par)
                start_rows(par, par)

    cc = _NCHUNK - 1
    wait_rows(0)
    wait_out(0)
    ob, _ = out_slots[0]
    _dot_chunk(sb0, db0, ob, dotsbuf, last_lane)
    start_out(cc, 0)
    wait_out(1)
    wait_out(0)


@jax.jit
def kernel(x, edge_index):
    mesh = plsc.VectorSubcoreMesh(core_axis_name="c", subcore_axis_name="s",
                                  num_cores=_NC, num_subcores=_NS)
    f = pl.kernel(
        _edge_kernel,
        out_type=jax.ShapeDtypeStruct((_E,), jnp.float32),
        mesh=mesh,
        compiler_params=pltpu.CompilerParams(needs_layout_passes=False),
        scratch_types=[
            pltpu.VMEM_SHARED((_N_NODES, _D), jnp.float32),
        ] + [
            pltpu.VMEM((_C, _D), jnp.float32)
            for _ in range(4)
        ] + [
            pltpu.VMEM((_C,), jnp.float32)
            for _ in range(2)
        ] + [
            pltpu.VMEM((_C,), jnp.float32),
        ] + [
            pltpu.VMEM((_C,), jnp.int32)
            for _ in range(2 * _NIDX)
        ] + [pltpu.SemaphoreType.DMA for _ in range(10)],
    )
    return f(x, edge_index.reshape(2 * _E))


# speedup vs baseline: 1.2246x; 1.2246x over previous
import jax
import jax.numpy as jnp
from jax import lax
from jax.experimental import pallas as pl
from jax.experimental.pallas import tpu as pltpu
from jax.experimental.pallas import tpu_sc as plsc

_N_NODES = 10000
_D = 128
_E = 320000
_NC = 2
_NS = 16
_NW = _NC * _NS
_EPW = _E // _NW
_C = 16
_NCHUNK = _EPW // _C


def _dot_chunk(sb, db, outv, dots, off, last_lane):
    for e in range(_C):
        p = sb[e, pl.ds(0, 16)] * db[e, pl.ds(0, 16)]
        for c in range(1, _D // 16):
            p = p + sb[e, pl.ds(c * 16, 16)] * db[e, pl.ds(c * 16, 16)]
        cum = plsc.cumsum(p)
        plsc.store_scatter(dots, [jnp.full((16,), e, jnp.int32)], cum,
                           mask=last_lane)
    v = dots[...]
    outv[pl.ds(off, 16)] = 1.0 / (1.0 + jnp.exp(-v))


def _edge_kernel(x_hbm, edge_hbm, out_hbm, x_s, sidx, didx,
                 sb0, db0, sb1, db1, outv, dots, ss0, sd0, ss1, sd1):
    cid = lax.axis_index("c")
    sid = lax.axis_index("s")
    wid = sid * _NC + cid
    base = wid * _EPW

    rows_per = 640

    @pl.when(sid < _NS - 1)
    def _():
        pltpu.sync_copy(x_hbm.at[pl.ds(sid * rows_per, rows_per)],
                        x_s.at[pl.ds(sid * rows_per, rows_per)])

    @pl.when(sid == _NS - 1)
    def _():
        last = (_NS - 1) * rows_per
        pltpu.sync_copy(x_hbm.at[pl.ds(last, _N_NODES - last)],
                        x_s.at[pl.ds(last, _N_NODES - last)])

    pltpu.sync_copy(edge_hbm.at[pl.ds(base, _EPW)], sidx)
    pltpu.sync_copy(edge_hbm.at[pl.ds(_E + base, _EPW)], didx)
    plsc.subcore_barrier()

    last_lane = lax.broadcasted_iota(jnp.int32, (16,), 0) == 15

    def start(cc, sb, db, ss, sd):
        o = cc * _C
        pltpu.async_copy(x_hbm.at[sidx.at[pl.ds(o, _C)]], sb, ss)
        pltpu.async_copy(x_s.at[didx.at[pl.ds(o, _C)]], db, sd)

    def wait(sb, db, ss, sd):
        pltpu.make_async_copy(x_hbm.at[pl.ds(0, _C)], sb, ss).wait()
        pltpu.make_async_copy(x_s.at[pl.ds(0, _C)], db, sd).wait()

    slots = ((sb0, db0, ss0, sd0), (sb1, db1, ss1, sd1))
    start(0, *slots[0])
    start(1, *slots[1])

    @pl.loop(0, _NCHUNK - 1, step=2)
    def _pair(c):
        for par in range(2):
            sb, db, ss, sd = slots[par]
            cc = c + par
            wait(sb, db, ss, sd)
            _dot_chunk(sb, db, outv, dots, cc * _C, last_lane)

            @pl.when(cc + 2 < _NCHUNK)
            def _(cc=cc, sb=sb, db=db, ss=ss, sd=sd):
                start(cc + 2, sb, db, ss, sd)

    wait(*slots[0])
    _dot_chunk(sb0, db0, outv, dots, (_NCHUNK - 1) * _C, last_lane)

    pltpu.sync_copy(outv, out_hbm.at[pl.ds(base, _EPW)])


@jax.jit
def kernel(x, edge_index):
    mesh = plsc.VectorSubcoreMesh(core_axis_name="c", subcore_axis_name="s",
                                  num_cores=_NC, num_subcores=_NS)
    f = pl.kernel(
        _edge_kernel,
        out_type=jax.ShapeDtypeStruct((_E,), jnp.float32),
        mesh=mesh,
        compiler_params=pltpu.CompilerParams(needs_layout_passes=False),
        scratch_types=[
            pltpu.VMEM_SHARED((_N_NODES, _D), jnp.float32),
            pltpu.VMEM((_EPW,), jnp.int32),
            pltpu.VMEM((_EPW,), jnp.int32),
            pltpu.VMEM((_C, _D), jnp.float32),
            pltpu.VMEM((_C, _D), jnp.float32),
            pltpu.VMEM((_C, _D), jnp.float32),
            pltpu.VMEM((_C, _D), jnp.float32),
            pltpu.VMEM((_EPW,), jnp.float32),
            pltpu.VMEM((16,), jnp.float32),
            pltpu.SemaphoreType.DMA,
            pltpu.SemaphoreType.DMA,
            pltpu.SemaphoreType.DMA,
            pltpu.SemaphoreType.DMA,
        ],
    )
    return f(x, edge_index.reshape(2 * _E))
